# baseline (speedup 1.0000x reference)
import jax
import jax.numpy as jnp
from jax.experimental import pallas as pl
from jax.experimental.pallas import tpu as pltpu

KEEP_RATIO = 0.5


def _scores_body(x_ref, o_ref, acc_ref):
    s = pl.program_id(2)
    ns = pl.num_programs(2)

    @pl.when(s == 0)
    def _init():
        acc_ref[...] = jnp.abs(x_ref[...])

    @pl.when(s > 0)
    def _accum():
        acc_ref[...] += jnp.abs(x_ref[...])

    @pl.when(s == ns - 1)
    def _reduce():
        o_ref[...] = jnp.sum(acc_ref[...], axis=(2, 3))


def _mask_body(s_ref, o_ref):
    b, c = s_ref.shape
    k = max(1, min(c, int(round(c * KEEP_RATIO))))
    bits = jax.lax.bitcast_convert_type(s_ref[...], jnp.int32)
    col = jax.lax.broadcasted_iota(jnp.int32, (b, c), 1)

    def vsearch(_, carry):
        lo, hi = carry
        mid = lo + ((hi - lo + 1) >> 1)
        cnt = jnp.sum((bits >= mid).astype(jnp.int32), axis=1, keepdims=True)
        p = cnt >= k
        return jnp.where(p, mid, lo), jnp.where(p, hi, mid - 1)

    lo = jnp.zeros((b, 1), jnp.int32)
    hi = jnp.full((b, 1), 0x7F800000, jnp.int32)
    t, _ = jax.lax.fori_loop(0, 31, vsearch, (lo, hi))

    gt = bits > t
    eq = bits == t
    need_eq = k - jnp.sum(gt.astype(jnp.int32), axis=1, keepdims=True)

    def isearch(_, carry):
        lo2, hi2 = carry
        mid = (lo2 + hi2) >> 1
        cnt = jnp.sum((eq & (col <= mid)).astype(jnp.int32), axis=1,
                      keepdims=True)
        p = cnt >= need_eq
        return jnp.where(p, lo2, mid + 1), jnp.where(p, mid, hi2)

    lo2 = jnp.zeros((b, 1), jnp.int32)
    hi2 = jnp.full((b, 1), c - 1, jnp.int32)
    m, _ = jax.lax.fori_loop(0, 10, isearch, (lo2, hi2))

    o_ref[...] = (gt | (eq & (col <= m))).astype(jnp.float32)


def kernel(features, enabled):
    b, c, h, w = features.shape

    bblk, cblk, hblk = 8, 128, 8
    scores = pl.pallas_call(
        _scores_body,
        grid=(b // bblk, c // cblk, h // hblk),
        in_specs=[pl.BlockSpec((bblk, cblk, hblk, w),
                               lambda i, j, s: (i, j, s, 0))],
        out_specs=pl.BlockSpec((bblk, cblk), lambda i, j, s: (i, j)),
        out_shape=jax.ShapeDtypeStruct((b, c), jnp.float32),
        scratch_shapes=[pltpu.VMEM((bblk, cblk, hblk, w), jnp.float32)],
        compiler_params=pltpu.CompilerParams(
            dimension_semantics=("parallel", "parallel", "arbitrary")),
    )(features)

    mask = pl.pallas_call(
        _mask_body,
        out_shape=jax.ShapeDtypeStruct((b, c), jnp.float32),
    )(scores)

    gated = mask.reshape(b, c, 1, 1)
    return jnp.where(jnp.asarray(enabled) != 0, gated,
                     jnp.ones_like(gated))


# speedup vs baseline: 1.0873x; 1.0873x over previous
import jax
import jax.numpy as jnp
from jax.experimental import pallas as pl
from jax.experimental.pallas import tpu as pltpu

KEEP_RATIO = 0.5


def _scores_body(x_hbm, o_ref, buf0, buf1, sem0, sem1):
    i = pl.program_id(0)
    n = pl.num_programs(0)

    @pl.when(i == 0)
    def _prime():
        pltpu.make_async_copy(x_hbm.at[0], buf0, sem0).start()

    @pl.when(jnp.logical_and(i + 1 < n, (i + 1) % 2 == 0))
    def _pf_even():
        pltpu.make_async_copy(x_hbm.at[i + 1], buf0, sem0).start()

    @pl.when(jnp.logical_and(i + 1 < n, (i + 1) % 2 == 1))
    def _pf_odd():
        pltpu.make_async_copy(x_hbm.at[i + 1], buf1, sem1).start()

    @pl.when(i % 2 == 0)
    def _even():
        pltpu.make_async_copy(x_hbm.at[i], buf0, sem0).wait()
        o_ref[0, 0, :] = jnp.sum(jnp.abs(buf0[...]), axis=(1, 2))

    @pl.when(i % 2 == 1)
    def _odd():
        pltpu.make_async_copy(x_hbm.at[i], buf1, sem1).wait()
        o_ref[0, 0, :] = jnp.sum(jnp.abs(buf1[...]), axis=(1, 2))


def _mask_body(s_ref, o_ref):
    b, c = s_ref.shape
    k = max(1, min(c, int(round(c * KEEP_RATIO))))
    bits = jax.lax.bitcast_convert_type(s_ref[...], jnp.int32)
    col = jax.lax.broadcasted_iota(jnp.int32, (b, c), 1)

    def vsearch(_, carry):
        lo, hi = carry
        mid = lo + ((hi - lo + 1) >> 1)
        cnt = jnp.sum((bits >= mid).astype(jnp.int32), axis=1, keepdims=True)
        p = cnt >= k
        return jnp.where(p, mid, lo), jnp.where(p, hi, mid - 1)

    lo = jnp.zeros((b, 1), jnp.int32)
    hi = jnp.full((b, 1), 0x7F800000, jnp.int32)
    t, _ = jax.lax.fori_loop(0, 31, vsearch, (lo, hi))

    gt = bits > t
    eq = bits == t
    need_eq = k - jnp.sum(gt.astype(jnp.int32), axis=1, keepdims=True)

    def isearch(_, carry):
        lo2, hi2 = carry
        mid = (lo2 + hi2) >> 1
        cnt = jnp.sum((eq & (col <= mid)).astype(jnp.int32), axis=1,
                      keepdims=True)
        p = cnt >= need_eq
        return jnp.where(p, lo2, mid + 1), jnp.where(p, mid, hi2)

    lo2 = jnp.zeros((b, 1), jnp.int32)
    hi2 = jnp.full((b, 1), c - 1, jnp.int32)
    m, _ = jax.lax.fori_loop(0, 10, isearch, (lo2, hi2))

    o_ref[...] = (gt | (eq & (col <= m))).astype(jnp.float32)


def kernel(features, enabled):
    b, c, h, w = features.shape

    scores3 = pl.pallas_call(
        _scores_body,
        grid=(b,),
        in_specs=[pl.BlockSpec(memory_space=pltpu.HBM)],
        out_specs=pl.BlockSpec((1, 1, c), lambda i: (i, 0, 0)),
        out_shape=jax.ShapeDtypeStruct((b, 1, c), jnp.float32),
        scratch_shapes=[
            pltpu.VMEM((c, h, w), jnp.float32),
            pltpu.VMEM((c, h, w), jnp.float32),
            pltpu.SemaphoreType.DMA,
            pltpu.SemaphoreType.DMA,
        ],
        compiler_params=pltpu.CompilerParams(
            dimension_semantics=("arbitrary",)),
    )(features)
    scores = scores3.reshape(b, c)

    mask = pl.pallas_call(
        _mask_body,
        out_shape=jax.ShapeDtypeStruct((b, c), jnp.float32),
    )(scores)

    gated = mask.reshape(b, c, 1, 1)
    return jnp.where(jnp.asarray(enabled) != 0, gated,
                     jnp.ones_like(gated))
